# Initial kernel scaffold; baseline (speedup 1.0000x reference)
import jax
import jax.numpy as jnp
from jax.experimental import pallas as pl


def kernel(inputs, Wx, bx, gx, bgx, Wy, by, gy, bgy, alpha, means, training):
    raise NotImplementedError("write your pallas kernel here")


# speedup vs baseline: 6.1357x; 6.1357x over previous
import functools

import jax
import jax.numpy as jnp
from jax import lax
from jax.experimental import pallas as pl
from jax.experimental.pallas import tpu as pltpu
from jax.experimental.pallas import tpu_sc as plsc

N_CLUSTERS = 128
N, L, C = 32, 4096, 64
CHUNK = 128
NCH = L // CHUNK

_ROLL = pltpu.roll


def _embed_math(x, W, b, g, bg):
    h = lax.dot_general(x, W, (((1,), (1,)), ((), ())),
                        preferred_element_type=jnp.float32) + b
    mu = jnp.mean(h, axis=-1, keepdims=True)
    var = jnp.mean((h - mu) ** 2, axis=-1, keepdims=True)
    h = (h - mu) / jnp.sqrt(var + 1e-5) * g + bg
    return jnp.maximum(h, 0.0)


def _k1_math(x, Wx, bx, gx, bgx, Wy, by, gy, bgy, means):
    xe = _embed_math(x, Wx, bx, gx, bgx)
    ye = _embed_math(x, Wy, by, gy, bgy)
    s = jnp.sqrt(jnp.sum(xe * xe, axis=0, keepdims=True))
    xn = xe / jnp.maximum(s, 1e-12)
    inv_m = 1.0 / jnp.maximum(s, 5e-5)
    d = lax.dot_general(xn, means, (((1,), (1,)), ((), ())),
                        preferred_element_type=jnp.float32)
    m = jnp.max(d, axis=-1, keepdims=True)
    ii = lax.broadcasted_iota(jnp.int32, d.shape, 1)
    codes = jnp.min(jnp.where(d == m, ii, N_CLUSTERS), axis=-1, keepdims=True)
    return xe, ye, codes, inv_m


def _k1_body(x_ref, wx_ref, bx_ref, gx_ref, bgx_ref,
             wy_ref, by_ref, gy_ref, bgy_ref, means_ref,
             xy_ref, codes_ref, inv_ref):
    xe, ye, codes, inv_m = _k1_math(
        x_ref[0], wx_ref[...], bx_ref[...], gx_ref[...], bgx_ref[...],
        wy_ref[...], by_ref[...], gy_ref[...], bgy_ref[...], means_ref[...])
    xy_ref[0] = jnp.concatenate([xe, ye], axis=-1)
    codes_ref[0] = codes
    inv_ref[0] = inv_m


def _run_k1(inputs, Wx, bx, gx, bgx, Wy, by, gy, bgy, means):
    full2 = lambda shp: pl.BlockSpec(shp, lambda b: (0, 0))
    return pl.pallas_call(
        _k1_body,
        grid=(N,),
        in_specs=[
            pl.BlockSpec((1, L, C), lambda b: (b, 0, 0)),
            full2((C, C)), full2((1, C)), full2((1, C)), full2((1, C)),
            full2((C, C)), full2((1, C)), full2((1, C)), full2((1, C)),
            full2((N_CLUSTERS, C)),
        ],
        out_specs=[
            pl.BlockSpec((1, L, 2 * C), lambda b: (b, 0, 0)),
            pl.BlockSpec((1, L, 1), lambda b: (b, 0, 0)),
            pl.BlockSpec((1, 1, C), lambda b: (b, 0, 0)),
        ],
        out_shape=[
            jax.ShapeDtypeStruct((N, L, 2 * C), jnp.float32),
            jax.ShapeDtypeStruct((N, L, 1), jnp.int32),
            jax.ShapeDtypeStruct((N, 1, C), jnp.float32),
        ],
    )(inputs, Wx, bx.reshape(1, C), gx.reshape(1, C), bgx.reshape(1, C),
      Wy, by.reshape(1, C), gy.reshape(1, C), bgy.reshape(1, C), means[0])


@functools.cache
def _sc_sort_gather_kernel():
    mesh = plsc.VectorSubcoreMesh(core_axis_name="c", subcore_axis_name="s")
    return functools.partial(
        pl.kernel,
        out_type=(
            jax.ShapeDtypeStruct((N, L), jnp.int32),
            jax.ShapeDtypeStruct((N, L, 2 * C), jnp.float32),
        ),
        mesh=mesh,
        scratch_types=(
            pltpu.VMEM((L,), jnp.int32),
            pltpu.VMEM((L,), jnp.int32),
            pltpu.VMEM((L,), jnp.int32),
            pltpu.VMEM((N_CLUSTERS,), jnp.int32),
            pltpu.VMEM((32,), jnp.int32),
            pltpu.VMEM((32,), jnp.int32),
            pltpu.VMEM((CHUNK, 2 * C), jnp.float32),
            pltpu.SemaphoreType.DMA,
        ),
        compiler_params=pltpu.CompilerParams(needs_layout_passes=False),
    )(_sc_sort_gather_body)


def _sc_sort_gather_body(codes_hbm, xy_hbm,
                         undo_hbm, xys_hbm,
                         codes_v, idx_v, undo_v, hist, scr_prev, scr_next,
                         xyrows, sem):
    b = lax.axis_index("s") * 2 + lax.axis_index("c")
    pltpu.sync_copy(codes_hbm.at[b], codes_v)

    iota = lax.broadcasted_iota(jnp.int32, (16,), 0)
    for i in range(N_CLUSTERS // 16):
        hist[pl.ds(i * 16, 16)] = jnp.zeros((16,), jnp.int32)
    scr_prev[pl.ds(0, 16)] = jnp.full((16,), -1, jnp.int32)
    scr_next[pl.ds(16, 16)] = jnp.ones((16,), jnp.int32)

    def seg_info(k):
        c = codes_v[pl.ds(k * 16, 16)]
        key = c * 16 + iota
        sk, sv = plsc.sort_key_val(key, iota)
        sc = lax.shift_right_logical(sk, 4)
        scr_prev[pl.ds(1, 16)] = sc
        prev = scr_prev[pl.ds(0, 16)]
        newseg = jnp.where(sc != prev, 1, 0)
        start = plsc.cummax(jnp.where(newseg > 0, iota, 0))
        rank = iota - start
        scr_next[pl.ds(0, 16)] = newseg
        nxt = scr_next[pl.ds(1, 16)]
        return sc, sv, rank, nxt > 0

    def histpass(k, carry):
        sc, _, rank, is_last = seg_info(k)
        plsc.addupdate_scatter(hist, [sc], rank + 1, mask=is_last)
        return carry
    lax.fori_loop(0, L // 16, histpass, 0)

    def prefix(i, run):
        h = hist[pl.ds(i * 16, 16)]
        inc = plsc.cumsum(h)
        hist[pl.ds(i * 16, 16)] = inc - h + run
        return run + jnp.sum(h)
    lax.fori_loop(0, N_CLUSTERS // 16, prefix, jnp.int32(0))

    def place(k, carry):
        sc, sv, rank, is_last = seg_info(k)
        cur = plsc.load_gather(hist, [sc])
        pos = cur + rank
        tok = sv + k * 16
        plsc.store_scatter(idx_v, [pos], tok)
        plsc.store_scatter(undo_v, [tok], pos)
        plsc.store_scatter(hist, [sc], pos + 1, mask=is_last)
        return carry
    lax.fori_loop(0, L // 16, place, 0)

    pltpu.sync_copy(undo_v, undo_hbm.at[b])

    def gather(j, carry):
        idxrow = idx_v.at[pl.ds(j * CHUNK, CHUNK)]
        pltpu.async_copy(xy_hbm.at[b].at[idxrow], xyrows, sem).wait()
        pltpu.sync_copy(xyrows, xys_hbm.at[b, pl.ds(j * CHUNK, CHUNK)])
        return carry
    lax.fori_loop(0, NCH, gather, 0)


def _k2_math(xs, ys, inv_m):
    xm = xs * inv_m
    xmp = _ROLL(xm, 1, 0)
    xmn = _ROLL(xm, L - 1, 0)
    r0 = jnp.sum(xs * xm, axis=-1, keepdims=True)
    r1 = jnp.sum(xs * xmp, axis=-1, keepdims=True)
    r2 = jnp.sum(xs * xmn, axis=-1, keepdims=True)
    m3 = jnp.maximum(jnp.maximum(r0, r1), r2)
    e0 = jnp.exp(r0 - m3)
    e1 = jnp.exp(r1 - m3)
    e2 = jnp.exp(r2 - m3)
    z = e0 + e1 + e2
    lse = m3 + jnp.log(z)
    yp = _ROLL(ys, 1, 0)
    yn = _ROLL(ys, L - 1, 0)
    ret = (e0 * ys + e1 * yp + e2 * yn) / z
    mm = jnp.max(lse, axis=0, keepdims=True)
    p = jnp.exp(lse - mm)
    ssum = jnp.sum(p, axis=0, keepdims=True)
    return ret * (p / ssum)


def _k2_body(xys_ref, inv_ref, out_ref):
    xy = xys_ref[0]
    out = _k2_math(xy[:, :C], xy[:, C:], inv_ref[0])
    out_ref[0] = jnp.concatenate([out, out], axis=-1)


def _run_k2(xys, inv_m):
    return pl.pallas_call(
        _k2_body,
        grid=(N,),
        in_specs=[
            pl.BlockSpec((1, L, 2 * C), lambda b: (b, 0, 0)),
            pl.BlockSpec((1, 1, C), lambda b: (b, 0, 0)),
        ],
        out_specs=pl.BlockSpec((1, L, 2 * C), lambda b: (b, 0, 0)),
        out_shape=jax.ShapeDtypeStruct((N, L, 2 * C), jnp.float32),
    )(xys, inv_m)


@functools.cache
def _sc_unsort_kernel():
    mesh = plsc.VectorSubcoreMesh(core_axis_name="c", subcore_axis_name="s")
    return functools.partial(
        pl.kernel,
        out_type=jax.ShapeDtypeStruct((N, L, 2 * C), jnp.float32),
        mesh=mesh,
        scratch_types=(
            pltpu.VMEM((L,), jnp.int32),
            pltpu.VMEM((CHUNK, 2 * C), jnp.float32),
            pltpu.SemaphoreType.DMA,
        ),
        compiler_params=pltpu.CompilerParams(needs_layout_passes=False),
    )(_sc_unsort_body)


def _sc_unsort_body(undo_hbm, os_hbm, out_hbm, undo_v, rows, sem):
    b = lax.axis_index("s") * 2 + lax.axis_index("c")
    pltpu.sync_copy(undo_hbm.at[b], undo_v)

    def gather(j, carry):
        idxrow = undo_v.at[pl.ds(j * CHUNK, CHUNK)]
        pltpu.async_copy(os_hbm.at[b].at[idxrow], rows, sem).wait()
        pltpu.sync_copy(rows, out_hbm.at[b, pl.ds(j * CHUNK, CHUNK)])
        return carry
    lax.fori_loop(0, NCH, gather, 0)


def kernel(inputs, Wx, bx, gx, bgx, Wy, by, gy, bgy, alpha, means, training):
    xy, codes3, inv_m = _run_k1(inputs, Wx, bx, gx, bgx, Wy, by, gy, bgy,
                                means)
    codes = codes3.reshape(N, L)
    undo, xys = _sc_sort_gather_kernel()(codes, xy)
    out_sorted = _run_k2(xys, inv_m)
    ret = _sc_unsort_kernel()(undo, out_sorted)
    af = alpha.astype(jnp.float32)
    return af * inputs + (1.0 - af) * ret[:, :, :C]


# speedup vs baseline: 6.4423x; 1.0500x over previous
import functools

import jax
import jax.numpy as jnp
from jax import lax
from jax.experimental import pallas as pl
from jax.experimental.pallas import tpu as pltpu
from jax.experimental.pallas import tpu_sc as plsc

N_CLUSTERS = 128
N, L, C = 32, 4096, 64
CHUNK = 128
NCH = L // CHUNK

_ROLL = pltpu.roll


def _ln_relu(h, g, bg):
    mu = jnp.mean(h, axis=-1, keepdims=True)
    var = jnp.mean((h - mu) ** 2, axis=-1, keepdims=True)
    return jnp.maximum((h - mu) / jnp.sqrt(var + 1e-5) * g + bg, 0.0)


def _k1_math(x, wcat, bcat, gcat, bgcat, means, eye):
    h = lax.dot_general(x, wcat, (((1,), (0,)), ((), ())),
                        preferred_element_type=jnp.float32) + bcat
    xe = _ln_relu(h[:, :C], gcat[:, :C], bgcat[:, :C])
    ye = _ln_relu(h[:, C:], gcat[:, C:], bgcat[:, C:])
    s = jnp.sqrt(jnp.sum(xe * xe, axis=0, keepdims=True))
    xn = xe / jnp.maximum(s, 1e-12)
    inv_m = 1.0 / jnp.maximum(s, 5e-5)
    d = lax.dot_general(xn, means, (((1,), (1,)), ((), ())),
                        preferred_element_type=jnp.float32)
    m = jnp.max(d, axis=-1, keepdims=True)
    ii = lax.broadcasted_iota(jnp.int32, d.shape, 1)
    codes = jnp.min(jnp.where(d == m, ii, N_CLUSTERS), axis=-1,
                    keepdims=True)
    cf = codes.astype(jnp.float32)
    rows = [lax.dot_general(cf[i * 128:(i + 1) * 128, :], eye,
                            (((0,), (0,)), ((), ())),
                            preferred_element_type=jnp.float32)
            for i in range(L // 128)]
    codes_row = jnp.concatenate(rows, axis=1).astype(jnp.int32)
    codes8 = jnp.broadcast_to(codes_row, (8, L))
    xy = jnp.concatenate([xe, ye], axis=-1)
    return xy, codes8, inv_m


def _k1_body(x_ref, wcat_ref, bcat_ref, gcat_ref, bgcat_ref,
             means_ref, eye_ref, xy_ref, codes_ref, inv_ref):
    xy, codes8, inv_m = _k1_math(
        x_ref[0], wcat_ref[...], bcat_ref[...], gcat_ref[...], bgcat_ref[...],
        means_ref[...], eye_ref[...])
    xy_ref[0] = xy
    codes_ref[0] = codes8
    inv_ref[0] = inv_m


def _run_k1(inputs, Wx, bx, gx, bgx, Wy, by, gy, bgy, means):
    wcat = jnp.concatenate([Wx.T, Wy.T], axis=1)
    bcat = jnp.concatenate([bx, by]).reshape(1, 2 * C)
    gcat = jnp.concatenate([gx, gy]).reshape(1, 2 * C)
    bgcat = jnp.concatenate([bgx, bgy]).reshape(1, 2 * C)
    eye = jnp.eye(128, dtype=jnp.float32)
    full2 = lambda shp: pl.BlockSpec(shp, lambda b: (0, 0))
    return pl.pallas_call(
        _k1_body,
        grid=(N,),
        in_specs=[
            pl.BlockSpec((1, L, C), lambda b: (b, 0, 0)),
            full2((C, 2 * C)), full2((1, 2 * C)), full2((1, 2 * C)),
            full2((1, 2 * C)), full2((N_CLUSTERS, C)), full2((128, 128)),
        ],
        out_specs=[
            pl.BlockSpec((1, L, 2 * C), lambda b: (b, 0, 0)),
            pl.BlockSpec((1, 8, L), lambda b: (b, 0, 0)),
            pl.BlockSpec((1, 1, C), lambda b: (b, 0, 0)),
        ],
        out_shape=[
            jax.ShapeDtypeStruct((N, L, 2 * C), jnp.float32),
            jax.ShapeDtypeStruct((N, 8, L), jnp.int32),
            jax.ShapeDtypeStruct((N, 1, C), jnp.float32),
        ],
    )(inputs, wcat, bcat, gcat, bgcat, means[0], eye)


@functools.cache
def _sc_sort_gather_kernel():
    mesh = plsc.VectorSubcoreMesh(core_axis_name="c", subcore_axis_name="s")
    return functools.partial(
        pl.kernel,
        out_type=(
            jax.ShapeDtypeStruct((N, L), jnp.int32),
            jax.ShapeDtypeStruct((N, L, 2 * C), jnp.float32),
        ),
        mesh=mesh,
        scratch_types=(
            pltpu.VMEM((L,), jnp.int32),
            pltpu.VMEM((L,), jnp.int32),
            pltpu.VMEM((L,), jnp.int32),
            pltpu.VMEM((N_CLUSTERS,), jnp.int32),
            pltpu.VMEM((32,), jnp.int32),
            pltpu.VMEM((32,), jnp.int32),
            pltpu.VMEM((CHUNK, 2 * C), jnp.float32),
            pltpu.SemaphoreType.DMA,
        ),
        compiler_params=pltpu.CompilerParams(needs_layout_passes=False),
    )(_sc_sort_gather_body)


def _sc_sort_gather_body(codes_hbm, xy_hbm,
                         undo_hbm, xys_hbm,
                         codes_v, idx_v, undo_v, hist, scr_prev, scr_next,
                         xyrows, sem):
    b = lax.axis_index("s") * 2 + lax.axis_index("c")
    pltpu.sync_copy(codes_hbm.at[b, 0], codes_v)

    iota = lax.broadcasted_iota(jnp.int32, (16,), 0)
    for i in range(N_CLUSTERS // 16):
        hist[pl.ds(i * 16, 16)] = jnp.zeros((16,), jnp.int32)
    scr_prev[pl.ds(0, 16)] = jnp.full((16,), -1, jnp.int32)
    scr_next[pl.ds(16, 16)] = jnp.ones((16,), jnp.int32)

    def seg_info(k):
        c = codes_v[pl.ds(k * 16, 16)]
        key = c * 16 + iota
        sk, sv = plsc.sort_key_val(key, iota)
        sc = lax.shift_right_logical(sk, 4)
        scr_prev[pl.ds(1, 16)] = sc
        prev = scr_prev[pl.ds(0, 16)]
        newseg = jnp.where(sc != prev, 1, 0)
        start = plsc.cummax(jnp.where(newseg > 0, iota, 0))
        rank = iota - start
        scr_next[pl.ds(0, 16)] = newseg
        nxt = scr_next[pl.ds(1, 16)]
        return sc, sv, rank, nxt > 0

    def histpass(k, carry):
        sc, _, rank, is_last = seg_info(k)
        plsc.addupdate_scatter(hist, [sc], rank + 1, mask=is_last)
        return carry
    lax.fori_loop(0, L // 16, histpass, 0)

    def prefix(i, run):
        h = hist[pl.ds(i * 16, 16)]
        inc = plsc.cumsum(h)
        hist[pl.ds(i * 16, 16)] = inc - h + run
        return run + jnp.sum(h)
    lax.fori_loop(0, N_CLUSTERS // 16, prefix, jnp.int32(0))

    def place(k, carry):
        sc, sv, rank, is_last = seg_info(k)
        cur = plsc.load_gather(hist, [sc])
        pos = cur + rank
        tok = sv + k * 16
        plsc.store_scatter(idx_v, [pos], tok)
        plsc.store_scatter(undo_v, [tok], pos)
        plsc.store_scatter(hist, [sc], pos + 1, mask=is_last)
        return carry
    lax.fori_loop(0, L // 16, place, 0)

    pltpu.sync_copy(undo_v, undo_hbm.at[b])

    def gather(j, carry):
        idxrow = idx_v.at[pl.ds(j * CHUNK, CHUNK)]
        pltpu.async_copy(xy_hbm.at[b].at[idxrow], xyrows, sem).wait()
        pltpu.sync_copy(xyrows, xys_hbm.at[b, pl.ds(j * CHUNK, CHUNK)])
        return carry
    lax.fori_loop(0, NCH, gather, 0)


def _k2_math(xy, inv_m):
    invp = jnp.concatenate([inv_m, jnp.zeros((1, C), jnp.float32)], axis=-1)
    xysi = xy * invp
    xyp = _ROLL(xy, 1, 0)
    ones_col = jnp.ones((2 * C, 1), jnp.float32)
    r0 = lax.dot_general(xy * xysi, ones_col, (((1,), (0,)), ((), ())),
                         preferred_element_type=jnp.float32)
    r1 = lax.dot_general(xyp * xysi, ones_col, (((1,), (0,)), ((), ())),
                         preferred_element_type=jnp.float32)
    r2 = _ROLL(r1, L - 1, 0)
    m3 = jnp.maximum(jnp.maximum(r0, r1), r2)
    mm = jnp.max(m3, axis=0, keepdims=True) + jnp.log(3.0).astype(jnp.float32)
    f0 = jnp.exp(r0 - mm)
    f1 = jnp.exp(r1 - mm)
    f2 = jnp.exp(r2 - mm)
    s = jnp.sum(f0 + f1 + f2, axis=0, keepdims=True)
    xyn = _ROLL(xy, L - 1, 0)
    acc = f0 * xy + f1 * xyp + f2 * xyn
    return acc * (1.0 / s)


def _k2_body(xys_ref, inv_ref, out_ref):
    out_ref[0] = _k2_math(xys_ref[0], inv_ref[0])


def _run_k2(xys, inv_m):
    return pl.pallas_call(
        _k2_body,
        grid=(N,),
        in_specs=[
            pl.BlockSpec((1, L, 2 * C), lambda b: (b, 0, 0)),
            pl.BlockSpec((1, 1, C), lambda b: (b, 0, 0)),
        ],
        out_specs=pl.BlockSpec((1, L, 2 * C), lambda b: (b, 0, 0)),
        out_shape=jax.ShapeDtypeStruct((N, L, 2 * C), jnp.float32),
    )(xys, inv_m)


@functools.cache
def _sc_unsort_kernel():
    mesh = plsc.VectorSubcoreMesh(core_axis_name="c", subcore_axis_name="s")
    return functools.partial(
        pl.kernel,
        out_type=jax.ShapeDtypeStruct((N, L, 2 * C), jnp.float32),
        mesh=mesh,
        scratch_types=(
            pltpu.VMEM((L,), jnp.int32),
            pltpu.VMEM((CHUNK, 2 * C), jnp.float32),
            pltpu.SemaphoreType.DMA,
        ),
        compiler_params=pltpu.CompilerParams(needs_layout_passes=False),
    )(_sc_unsort_body)


def _sc_unsort_body(undo_hbm, os_hbm, out_hbm, undo_v, rows, sem):
    b = lax.axis_index("s") * 2 + lax.axis_index("c")
    pltpu.sync_copy(undo_hbm.at[b], undo_v)

    def gather(j, carry):
        idxrow = undo_v.at[pl.ds(j * CHUNK, CHUNK)]
        pltpu.async_copy(os_hbm.at[b].at[idxrow], rows, sem).wait()
        pltpu.sync_copy(rows, out_hbm.at[b, pl.ds(j * CHUNK, CHUNK)])
        return carry
    lax.fori_loop(0, NCH, gather, 0)


def kernel(inputs, Wx, bx, gx, bgx, Wy, by, gy, bgy, alpha, means, training):
    xy, codes8, inv_m = _run_k1(inputs, Wx, bx, gx, bgx, Wy, by, gy, bgy,
                                means)
    undo, xys = _sc_sort_gather_kernel()(codes8, xy)
    out_sorted = _run_k2(xys, inv_m)
    ret = _sc_unsort_kernel()(undo, out_sorted)
    af = alpha.astype(jnp.float32)
    return af * inputs + (1.0 - af) * ret[:, :, C:]
